# baseline (speedup 1.0000x reference)
import functools

import jax
import jax.numpy as jnp
from jax import lax
from jax.experimental import pallas as pl
from jax.experimental.pallas import tpu as pltpu
from jax.experimental.pallas import tpu_sc as plsc

_NUM_TOK = 8192
_IN_DIM = 4096
_NUM_EXPERTS = 64
_D_E = 64
_TOP_K = 8
_BLK = 1024

_NW = 32
_CHUNKS = 2


def _gates_block(tau_ref, h_ref, w_ref, ee_ref, fg_ref):
    f32 = jnp.float32
    hp = jax.lax.dot_general(
        h_ref[...], w_ref[...], (((1,), (1,)), ((), ())),
        preferred_element_type=f32, precision=jax.lax.Precision.DEFAULT)
    hpt = hp.T
    nrm = jnp.sqrt(jnp.sum(hpt * hpt, axis=0, keepdims=True))
    hnt = hpt / jnp.maximum(nrm, 1e-12)
    ee = ee_ref[...]
    een = ee / jnp.maximum(
        jnp.sqrt(jnp.sum(ee * ee, axis=-1, keepdims=True)), 1e-12)
    scores = jax.lax.dot_general(
        een, hnt, (((1,), (0,)), ((), ())),
        preferred_element_type=f32, precision=jax.lax.Precision.DEFAULT)
    x = scores / tau_ref[0]
    m = jnp.max(x, axis=0, keepdims=True)
    ex = jnp.exp(x - m)
    fg = ex / jnp.sum(ex, axis=0, keepdims=True)
    fg_ref[...] = fg.T


def _tc_gates_chunk(h, W, expert_embeddings, tau, ci, n_tok):
    grid = (n_tok // _BLK,)
    blk0 = ci * (n_tok // _BLK)
    return pl.pallas_call(
        _gates_block,
        grid=grid,
        in_specs=[
            pl.BlockSpec(memory_space=pltpu.SMEM),
            pl.BlockSpec((_BLK, _IN_DIM), lambda i: (blk0 + i, 0)),
            pl.BlockSpec((_D_E, _IN_DIM), lambda i: (0, 0)),
            pl.BlockSpec((_NUM_EXPERTS, _D_E), lambda i: (0, 0)),
        ],
        out_specs=pl.BlockSpec((_BLK, _NUM_EXPERTS), lambda i: (i, 0)),
        out_shape=jax.ShapeDtypeStruct((n_tok, _NUM_EXPERTS), jnp.float32),
        compiler_params=pltpu.CompilerParams(
            dimension_semantics=("arbitrary",),
        ),
    )(jnp.reshape(tau, (1,)), h, W, expert_embeddings)


def _merge(va, ia, vb, ib):
    upd = (vb > va) | ((vb == va) & (ib < ia))
    return jnp.where(upd, vb, va), jnp.where(upd, ib, ia)


def _sc_route_body(fg_hbm, sg_hbm, idx_hbm, fg_v, sg_v, idx_v, *, sc_rows):
    i32 = jnp.int32
    f32 = jnp.float32
    wid = lax.axis_index("s") * 2 + lax.axis_index("c")
    base = wid * sc_rows
    pltpu.sync_copy(fg_hbm.at[pl.ds(base * _NUM_EXPERTS,
                                    sc_rows * _NUM_EXPERTS)], fg_v)
    lanes = lax.iota(i32, 16)
    zero16 = jnp.zeros((16,), f32)
    msk8 = lanes < _TOP_K

    def pmerge(a, b):
        ak, av = a
        bkr = lax.rev(b[0], (0,))
        bvr = lax.rev(b[1], (0,))
        ta = ak >= bkr
        hk = jnp.where(ta, ak, bkr)
        hv = jnp.where(ta, av, bvr)
        return plsc.sort_key_val(hk, hv, descending=True)

    def row_body(r, carry):
        off = r * _NUM_EXPERTS
        chunks = []
        for c in range(_NUM_EXPERTS // 16):
            keys = fg_v[pl.ds(off + c * 16, 16)]
            chunks.append(
                plsc.sort_key_val(keys, lanes + c * 16, descending=True))
        k16, v16 = pmerge(pmerge(chunks[0], chunks[1]),
                          pmerge(chunks[2], chunks[3]))
        ev = jnp.where(msk8, jnp.exp(k16), 0.0)
        nt = ev / jnp.sum(ev)
        for c in range(_NUM_EXPERTS // 16):
            sg_v[pl.ds(off + c * 16, 16)] = zero16
        plsc.store_scatter(sg_v, [off + v16], nt, mask=msk8)
        plsc.store_scatter(idx_v, [r * _TOP_K + lanes], v16, mask=msk8)
        return carry

    lax.fori_loop(0, sc_rows, row_body, 0, unroll=4)
    pltpu.sync_copy(sg_v, sg_hbm.at[pl.ds(base * _NUM_EXPERTS,
                                          sc_rows * _NUM_EXPERTS)])
    pltpu.sync_copy(idx_v, idx_hbm.at[pl.ds(base * _TOP_K,
                                            sc_rows * _TOP_K)])


@functools.cache
def _sc_route(n_tok):
    sc_rows = n_tok // _NW
    return pl.kernel(
        functools.partial(_sc_route_body, sc_rows=sc_rows),
        out_type=[
            jax.ShapeDtypeStruct((n_tok * _NUM_EXPERTS,), jnp.float32),
            jax.ShapeDtypeStruct((n_tok * _TOP_K,), jnp.int32),
        ],
        mesh=plsc.VectorSubcoreMesh(core_axis_name="c", subcore_axis_name="s"),
        scratch_types=[
            pltpu.VMEM((sc_rows * _NUM_EXPERTS,), jnp.float32),
            pltpu.VMEM((sc_rows * _NUM_EXPERTS,), jnp.float32),
            pltpu.VMEM((sc_rows * _TOP_K,), jnp.int32),
        ],
        compiler_params=pltpu.CompilerParams(needs_layout_passes=False),
    )


@jax.jit
def _router(h, W, expert_embeddings, tau):
    n_tok = _NUM_TOK // _CHUNKS
    fgs, sgs, idxs = [], [], []
    for ci in range(_CHUNKS):
        fg = _tc_gates_chunk(h, W, expert_embeddings, tau, ci, n_tok)
        sg_flat, idx_flat = _sc_route(n_tok)(jnp.reshape(fg, (-1,)))
        fgs.append(fg)
        sgs.append(jnp.reshape(sg_flat, (n_tok, _NUM_EXPERTS)))
        idxs.append(jnp.reshape(idx_flat, (n_tok, _TOP_K)))
    return (jnp.concatenate(sgs, axis=0),
            jnp.concatenate(idxs, axis=0),
            jnp.concatenate(fgs, axis=0))


def kernel(h, W, expert_embeddings, tau):
    return _router(h, W, expert_embeddings, tau)


# speedup vs baseline: 1.0020x; 1.0020x over previous
import functools

import jax
import jax.numpy as jnp
from jax import lax
from jax.experimental import pallas as pl
from jax.experimental.pallas import tpu as pltpu
from jax.experimental.pallas import tpu_sc as plsc

_NUM_TOK = 8192
_IN_DIM = 4096
_NUM_EXPERTS = 64
_D_E = 64
_TOP_K = 8
_BLK = 1024

_NW = 32
_CHUNKS = 2


def _gates_block(tau_ref, h_ref, w_ref, ee_ref, fg_ref):
    f32 = jnp.float32
    hp = jax.lax.dot_general(
        h_ref[...], w_ref[...], (((1,), (1,)), ((), ())),
        preferred_element_type=f32, precision=jax.lax.Precision.DEFAULT)
    hpt = hp.T
    nrm = jnp.sqrt(jnp.sum(hpt * hpt, axis=0, keepdims=True))
    hnt = hpt / jnp.maximum(nrm, 1e-12)
    ee = ee_ref[...]
    een = ee / jnp.maximum(
        jnp.sqrt(jnp.sum(ee * ee, axis=-1, keepdims=True)), 1e-12)
    scores = jax.lax.dot_general(
        een, hnt, (((1,), (0,)), ((), ())),
        preferred_element_type=f32, precision=jax.lax.Precision.DEFAULT)
    x = scores / tau_ref[0]
    m = jnp.max(x, axis=0, keepdims=True)
    ex = jnp.exp(x - m)
    fg = ex / jnp.sum(ex, axis=0, keepdims=True)
    fg_ref[...] = fg.T


def _tc_gates_chunk(h, W, expert_embeddings, tau, ci, n_tok):
    grid = (n_tok // _BLK,)
    blk0 = ci * (n_tok // _BLK)
    return pl.pallas_call(
        _gates_block,
        grid=grid,
        in_specs=[
            pl.BlockSpec(memory_space=pltpu.SMEM),
            pl.BlockSpec((_BLK, _IN_DIM), lambda i: (blk0 + i, 0)),
            pl.BlockSpec((_D_E, _IN_DIM), lambda i: (0, 0)),
            pl.BlockSpec((_NUM_EXPERTS, _D_E), lambda i: (0, 0)),
        ],
        out_specs=pl.BlockSpec((_BLK, _NUM_EXPERTS), lambda i: (i, 0)),
        out_shape=jax.ShapeDtypeStruct((n_tok, _NUM_EXPERTS), jnp.float32),
        compiler_params=pltpu.CompilerParams(
            dimension_semantics=("arbitrary",),
        ),
    )(jnp.reshape(tau, (1,)), h, W, expert_embeddings)


def _merge(va, ia, vb, ib):
    upd = (vb > va) | ((vb == va) & (ib < ia))
    return jnp.where(upd, vb, va), jnp.where(upd, ib, ia)


def _sc_route_body(fg_hbm, sg_hbm, idx_hbm, fg_v, sg_v, idx_v, *, sc_rows):
    i32 = jnp.int32
    f32 = jnp.float32
    wid = lax.axis_index("s") * 2 + lax.axis_index("c")
    base = wid * sc_rows
    pltpu.sync_copy(fg_hbm.at[pl.ds(base * _NUM_EXPERTS,
                                    sc_rows * _NUM_EXPERTS)], fg_v)
    lanes = lax.iota(i32, 16)
    zero16 = jnp.zeros((16,), f32)
    msk8 = lanes < _TOP_K

    def pmerge(a, b):
        ak, av = a
        bkr = lax.rev(b[0], (0,))
        bvr = lax.rev(b[1], (0,))
        ta = ak >= bkr
        hk = jnp.where(ta, ak, bkr)
        hv = jnp.where(ta, av, bvr)
        return plsc.sort_key_val(hk, hv, descending=True)

    def row_body(r, carry):
        off = r * _NUM_EXPERTS
        chunks = []
        for c in range(_NUM_EXPERTS // 16):
            keys = fg_v[pl.ds(off + c * 16, 16)]
            chunks.append(
                plsc.sort_key_val(keys, lanes + c * 16, descending=True))
        k16, v16 = pmerge(pmerge(chunks[0], chunks[1]),
                          pmerge(chunks[2], chunks[3]))
        ev = jnp.where(msk8, jnp.exp(k16), 0.0)
        nt = ev / jnp.sum(ev)
        for c in range(_NUM_EXPERTS // 16):
            sg_v[pl.ds(off + c * 16, 16)] = zero16
        plsc.store_scatter(sg_v, [off + v16], nt, mask=msk8)
        plsc.store_scatter(idx_v, [r * _TOP_K + lanes], v16, mask=msk8)
        return carry

    lax.fori_loop(0, sc_rows, row_body, 0, unroll=4)
    pltpu.sync_copy(sg_v, sg_hbm.at[pl.ds(base * _NUM_EXPERTS,
                                          sc_rows * _NUM_EXPERTS)])
    pltpu.sync_copy(idx_v, idx_hbm.at[pl.ds(base * _TOP_K,
                                            sc_rows * _TOP_K)])


@functools.cache
def _sc_route(n_tok):
    sc_rows = n_tok // _NW
    return pl.kernel(
        functools.partial(_sc_route_body, sc_rows=sc_rows),
        out_type=[
            jax.ShapeDtypeStruct((n_tok * _NUM_EXPERTS,), jnp.float32),
            jax.ShapeDtypeStruct((n_tok * _TOP_K,), jnp.int32),
        ],
        mesh=plsc.VectorSubcoreMesh(core_axis_name="c", subcore_axis_name="s"),
        scratch_types=[
            pltpu.VMEM((sc_rows * _NUM_EXPERTS,), jnp.float32),
            pltpu.VMEM((sc_rows * _NUM_EXPERTS,), jnp.float32),
            pltpu.VMEM((sc_rows * _TOP_K,), jnp.int32),
        ],
        compiler_params=pltpu.CompilerParams(needs_layout_passes=False),
    )


@jax.jit
def _router(h, W, expert_embeddings, tau):
    n_tok = _NUM_TOK // _CHUNKS
    fgs, sgs, idxs = [], [], []
    for ci in range(_CHUNKS):
        fgs.append(_tc_gates_chunk(h, W, expert_embeddings, tau, ci, n_tok))
    for ci in range(_CHUNKS):
        sg_flat, idx_flat = _sc_route(n_tok)(jnp.reshape(fgs[ci], (-1,)))
        sgs.append(jnp.reshape(sg_flat, (n_tok, _NUM_EXPERTS)))
        idxs.append(jnp.reshape(idx_flat, (n_tok, _TOP_K)))
    return (jnp.concatenate(sgs, axis=0),
            jnp.concatenate(idxs, axis=0),
            jnp.concatenate(fgs, axis=0))


def kernel(h, W, expert_embeddings, tau):
    return _router(h, W, expert_embeddings, tau)


# speedup vs baseline: 1.1627x; 1.1604x over previous
import functools

import jax
import jax.numpy as jnp
from jax import lax
from jax.experimental import pallas as pl
from jax.experimental.pallas import tpu as pltpu
from jax.experimental.pallas import tpu_sc as plsc

_NUM_TOK = 8192
_IN_DIM = 4096
_NUM_EXPERTS = 64
_D_E = 64
_TOP_K = 8
_BLK = 1024

_NW = 32


def _router_block(tau_ref, h_ref, w_ref, ee_ref, fg_ref, idx_ref, nt_ref):
    f32 = jnp.float32
    hp = jax.lax.dot_general(
        h_ref[...], w_ref[...], (((1,), (1,)), ((), ())),
        preferred_element_type=f32, precision=jax.lax.Precision.DEFAULT)
    hpt = hp.T
    nrm = jnp.sqrt(jnp.sum(hpt * hpt, axis=0, keepdims=True))
    hnt = hpt / jnp.maximum(nrm, 1e-12)
    ee = ee_ref[...]
    een = ee / jnp.maximum(
        jnp.sqrt(jnp.sum(ee * ee, axis=-1, keepdims=True)), 1e-12)
    scores = jax.lax.dot_general(
        een, hnt, (((1,), (0,)), ((), ())),
        preferred_element_type=f32, precision=jax.lax.Precision.DEFAULT)
    x = scores / tau_ref[0]
    m = jnp.max(x, axis=0, keepdims=True)
    ex = jnp.exp(x - m)
    fg = ex / jnp.sum(ex, axis=0, keepdims=True)
    fg_ref[...] = fg.T

    iota_e = jax.lax.broadcasted_iota(jnp.int32, fg.shape, 0)
    iota_k = jax.lax.broadcasted_iota(jnp.int32, (_TOP_K, fg.shape[1]), 0)
    work = fg
    vals = jnp.zeros((_TOP_K, fg.shape[1]), f32)
    idxs = jnp.zeros((_TOP_K, fg.shape[1]), jnp.int32)
    for k in range(_TOP_K):
        v = jnp.max(work, axis=0, keepdims=True)
        i = jnp.min(jnp.where(work == v, iota_e, _NUM_EXPERTS),
                    axis=0, keepdims=True)
        vals = jnp.where(iota_k == k, v, vals)
        idxs = jnp.where(iota_k == k, i, idxs)
        work = jnp.where(iota_e == i, -1.0, work)
    idx_ref[...] = idxs.T

    ev = jnp.exp(vals - jax.lax.slice_in_dim(vals, 0, 1, axis=0))
    nt_ref[...] = (ev / jnp.sum(ev, axis=0, keepdims=True)).T


def _tc_router(h, W, expert_embeddings, tau):
    grid = (_NUM_TOK // _BLK,)
    return pl.pallas_call(
        _router_block,
        grid=grid,
        in_specs=[
            pl.BlockSpec(memory_space=pltpu.SMEM),
            pl.BlockSpec((_BLK, _IN_DIM), lambda i: (i, 0)),
            pl.BlockSpec((_D_E, _IN_DIM), lambda i: (0, 0)),
            pl.BlockSpec((_NUM_EXPERTS, _D_E), lambda i: (0, 0)),
        ],
        out_specs=[
            pl.BlockSpec((_BLK, _NUM_EXPERTS), lambda i: (i, 0)),
            pl.BlockSpec((_BLK, _TOP_K), lambda i: (i, 0)),
            pl.BlockSpec((_BLK, _TOP_K), lambda i: (i, 0)),
        ],
        out_shape=[
            jax.ShapeDtypeStruct((_NUM_TOK, _NUM_EXPERTS), jnp.float32),
            jax.ShapeDtypeStruct((_NUM_TOK, _TOP_K), jnp.int32),
            jax.ShapeDtypeStruct((_NUM_TOK, _TOP_K), jnp.float32),
        ],
        compiler_params=pltpu.CompilerParams(
            dimension_semantics=("arbitrary",),
        ),
    )(jnp.reshape(tau, (1,)), h, W, expert_embeddings)


def _sc_scatter_body(idx_hbm, nt_hbm, sg_hbm, idx_v, nt_v, sg_v, *, sc_rows):
    i32 = jnp.int32
    wid = lax.axis_index("s") * 2 + lax.axis_index("c")
    base = wid * sc_rows
    pltpu.sync_copy(idx_hbm.at[pl.ds(base * _TOP_K, sc_rows * _TOP_K)], idx_v)
    pltpu.sync_copy(nt_hbm.at[pl.ds(base * _TOP_K, sc_rows * _TOP_K)], nt_v)
    lanes = lax.iota(i32, 16)
    zero16 = jnp.zeros((16,), jnp.float32)

    def zero_body(j, carry):
        sg_v[pl.ds(j * 16, 16)] = zero16
        return carry

    lax.fori_loop(0, sc_rows * _NUM_EXPERTS // 16, zero_body, 0, unroll=8)

    def group_body(g, carry):
        rows8 = (g * 16 + lanes) * _TOP_K
        rows64 = (g * 16 + lanes) * _NUM_EXPERTS
        for k in range(_TOP_K):
            iv = plsc.load_gather(idx_v, [rows8 + k])
            nv = plsc.load_gather(nt_v, [rows8 + k])
            plsc.store_scatter(sg_v, [rows64 + iv], nv)
        return carry

    lax.fori_loop(0, sc_rows // 16, group_body, 0, unroll=2)
    pltpu.sync_copy(sg_v, sg_hbm.at[pl.ds(base * _NUM_EXPERTS,
                                          sc_rows * _NUM_EXPERTS)])


@functools.cache
def _sc_scatter(n_tok):
    sc_rows = n_tok // _NW
    return pl.kernel(
        functools.partial(_sc_scatter_body, sc_rows=sc_rows),
        out_type=jax.ShapeDtypeStruct((n_tok * _NUM_EXPERTS,), jnp.float32),
        mesh=plsc.VectorSubcoreMesh(core_axis_name="c", subcore_axis_name="s"),
        scratch_types=[
            pltpu.VMEM((sc_rows * _TOP_K,), jnp.int32),
            pltpu.VMEM((sc_rows * _TOP_K,), jnp.float32),
            pltpu.VMEM((sc_rows * _NUM_EXPERTS,), jnp.float32),
        ],
        compiler_params=pltpu.CompilerParams(needs_layout_passes=False),
    )


@jax.jit
def _router(h, W, expert_embeddings, tau):
    fg, idx, nt = _tc_router(h, W, expert_embeddings, tau)
    sg_flat = _sc_scatter(_NUM_TOK)(jnp.reshape(idx, (-1,)),
                                    jnp.reshape(nt, (-1,)))
    sg = jnp.reshape(sg_flat, (_NUM_TOK, _NUM_EXPERTS))
    return sg, idx, fg


def kernel(h, W, expert_embeddings, tau):
    return _router(h, W, expert_embeddings, tau)
